# baseline (speedup 1.0000x reference)
import functools

import jax
import jax.numpy as jnp
from jax import lax
from jax.experimental import pallas as pl
from jax.experimental.pallas import tpu as pltpu
from jax.experimental.pallas import tpu_sc as plsc

N_NODES = 50000
N_EDGES = 1600000
NUM_WORKERS = 32
CHUNK = 3200
NCHUNKS = N_EDGES // CHUNK
MAX_G = -(-NCHUNKS // NUM_WORKERS)
VECS = CHUNK // 16
ACC_STRIDE = 1025
NSUB_PAD = 1024
LANES = 16

_mesh = plsc.VectorSubcoreMesh(core_axis_name="c", subcore_axis_name="s")


@functools.partial(
    pl.kernel,
    mesh=_mesh,
    compiler_params=pltpu.CompilerParams(needs_layout_passes=False),
    out_type=jax.ShapeDtypeStruct((NUM_WORKERS, 8, 128), jnp.float32),
    scratch_types=[
        pltpu.VMEM((N_NODES,), jnp.int32),
        pltpu.VMEM((2, CHUNK), jnp.int32),
        pltpu.VMEM((2, CHUNK), jnp.int32),
        pltpu.VMEM((CHUNK,), jnp.float32),
        pltpu.VMEM((CHUNK,), jnp.float32),
        pltpu.VMEM((NSUB_PAD,), jnp.float32),
        pltpu.VMEM((NSUB_PAD,), jnp.float32),
        pltpu.VMEM((8, 128), jnp.float32),
        pltpu.SemaphoreType.DMA,
        pltpu.SemaphoreType.DMA,
        pltpu.SemaphoreType.DMA,
        pltpu.SemaphoreType.DMA,
    ],
)
def _sc_scatter(w_hbm, ei_hbm, packed_hbm, out_hbm,
                packed_v, eb0, eb1, wb0, wb1, acc_a, acc_b, red_v,
                sem_i0, sem_i1, sem_w0, sem_w1):
  wid = lax.axis_index("c") * 16 + lax.axis_index("s")

  bufs = [(eb0, wb0, sem_i0, sem_w0), (eb1, wb1, sem_i1, sem_w1)]

  def chunk_off(g):
    cid = wid + NUM_WORKERS * g
    cid = jnp.minimum(cid, NCHUNKS - 1)
    return pl.multiple_of(cid * CHUNK, 128)

  def start_dma(g, eb, wbuf, si, sw):
    off = chunk_off(g)
    pltpu.make_async_copy(ei_hbm.at[:, pl.ds(off, CHUNK)], eb, si).start()
    pltpu.make_async_copy(w_hbm.at[pl.ds(off, CHUNK)], wbuf, sw).start()

  start_dma(0, eb0, wb0, sem_i0, sem_w0)
  pltpu.sync_copy(packed_hbm, packed_v)

  zeros16 = jnp.zeros((16,), jnp.float32)

  @plsc.parallel_loop(0, NSUB_PAD // 16)
  def _(i):
    sl = pl.ds(pl.multiple_of(i * 16, 16), 16)
    acc_a[sl] = zeros16
    acc_b[sl] = zeros16

  lomask = jnp.full((16,), 0xFFFF, jnp.int32)
  himask = jnp.full((16,), -65536, jnp.int32)

  def process(eb, wbuf):
    @plsc.parallel_loop(0, VECS // 2, unroll=8)
    def _(h):
      for k, acc in ((0, acc_a), (1, acc_b)):
        sl = pl.ds(pl.multiple_of((h * 2 + k) * 16, 16), 16)
        a = eb[1, sl]
        w = wbuf[sl]
        packed = plsc.load_gather(packed_v, [a])
        d = plsc.bitcast(packed & himask, jnp.float32)
        s = packed & lomask
        plsc.addupdate_scatter(acc, [s], w * d)

  for g in range(MAX_G):
    eb, wbuf, si, sw = bufs[g % 2]
    pltpu.make_async_copy(ei_hbm.at[:, pl.ds(chunk_off(g), CHUNK)], eb, si).wait()
    pltpu.make_async_copy(w_hbm.at[pl.ds(chunk_off(g), CHUNK)], wbuf, sw).wait()
    if g + 1 < MAX_G:
      neb, nwb, nsi, nsw = bufs[(g + 1) % 2]
      start_dma(g + 1, neb, nwb, nsi, nsw)
    if g + 1 < MAX_G:
      process(eb, wbuf)
    else:
      @pl.when(wid + NUM_WORKERS * g < NCHUNKS)
      def _():
        process(eb, wbuf)

  @plsc.parallel_loop(0, NSUB_PAD // 16)
  def _(c):
    sl = pl.ds(pl.multiple_of(c * 16, 16), 16)
    s = acc_a[sl] + acc_b[sl]
    red_v[c // 8, pl.ds(pl.multiple_of((c % 8) * 16, 16), 16)] = s

  pltpu.sync_copy(red_v, out_hbm.at[wid])


def _combine_body(p_ref, y_ref, o_ref, *, nsub):
  s = jnp.sum(p_ref[...], axis=0)
  r = jnp.abs(s - y_ref[...])
  idx = (lax.broadcasted_iota(jnp.int32, (8, 128), 0) * 128
         + lax.broadcasted_iota(jnp.int32, (8, 128), 1))
  r = jnp.where(idx < nsub, r, 0.0)
  loss = jnp.sum(r) * (1.0 / nsub)
  o_ref[...] = jnp.zeros((8, 128), jnp.float32) + loss


def kernel(edge_weights, edge_index, agent_demand, agent_substation_map,
           substation_y_true, num_substations):
  submap = agent_substation_map.astype(jnp.uint32)
  nsub = substation_y_true.shape[0]

  d16 = lax.bitcast_convert_type(
      agent_demand.astype(jnp.bfloat16), jnp.uint16).astype(jnp.uint32)
  packed = lax.bitcast_convert_type((d16 << 16) | submap, jnp.int32)

  partials = _sc_scatter(edge_weights, edge_index.astype(jnp.int32), packed)

  y_pad = jnp.pad(substation_y_true, (0, NSUB_PAD - nsub)).reshape(8, 128)
  loss = pl.pallas_call(
      functools.partial(_combine_body, nsub=nsub),
      out_shape=jax.ShapeDtypeStruct((8, 128), jnp.float32),
  )(partials, y_pad)
  return loss[0, 0]


# speedup vs baseline: 1.1752x; 1.1752x over previous
import functools

import jax
import jax.numpy as jnp
from jax import lax
from jax.experimental import pallas as pl
from jax.experimental.pallas import tpu as pltpu
from jax.experimental.pallas import tpu_sc as plsc

N_NODES = 50000
N_EDGES = 1600000
NUM_WORKERS = 32
CHUNK = 12800
NCHUNKS = N_EDGES // CHUNK
MAX_G = -(-NCHUNKS // NUM_WORKERS)
VECS = CHUNK // 16
ACC_STRIDE = 1025
NSUB_PAD = 1024
LANES = 16

_mesh = plsc.VectorSubcoreMesh(core_axis_name="c", subcore_axis_name="s")


@functools.partial(
    pl.kernel,
    mesh=_mesh,
    compiler_params=pltpu.CompilerParams(needs_layout_passes=False),
    out_type=jax.ShapeDtypeStruct((NUM_WORKERS, 8, 128), jnp.float32),
    scratch_types=[
        pltpu.VMEM((N_NODES,), jnp.int32),
        pltpu.VMEM((2, CHUNK), jnp.int32),
        pltpu.VMEM((2, CHUNK), jnp.int32),
        pltpu.VMEM((CHUNK,), jnp.float32),
        pltpu.VMEM((CHUNK,), jnp.float32),
        pltpu.VMEM((NSUB_PAD,), jnp.float32),
        pltpu.VMEM((NSUB_PAD,), jnp.float32),
        pltpu.VMEM((8, 128), jnp.float32),
        pltpu.SemaphoreType.DMA,
        pltpu.SemaphoreType.DMA,
        pltpu.SemaphoreType.DMA,
        pltpu.SemaphoreType.DMA,
    ],
)
def _sc_scatter(w_hbm, ei_hbm, packed_hbm, out_hbm,
                packed_v, eb0, eb1, wb0, wb1, acc_a, acc_b, red_v,
                sem_i0, sem_i1, sem_w0, sem_w1):
  wid = lax.axis_index("c") * 16 + lax.axis_index("s")

  bufs = [(eb0, wb0, sem_i0, sem_w0), (eb1, wb1, sem_i1, sem_w1)]

  def chunk_off(g):
    cid = wid + NUM_WORKERS * g
    cid = jnp.minimum(cid, NCHUNKS - 1)
    return pl.multiple_of(cid * CHUNK, 128)

  def start_dma(g, eb, wbuf, si, sw):
    off = chunk_off(g)
    pltpu.make_async_copy(ei_hbm.at[:, pl.ds(off, CHUNK)], eb, si).start()
    pltpu.make_async_copy(w_hbm.at[pl.ds(off, CHUNK)], wbuf, sw).start()

  start_dma(0, eb0, wb0, sem_i0, sem_w0)
  pltpu.sync_copy(packed_hbm, packed_v)

  zeros16 = jnp.zeros((16,), jnp.float32)

  @plsc.parallel_loop(0, NSUB_PAD // 16)
  def _(i):
    sl = pl.ds(pl.multiple_of(i * 16, 16), 16)
    acc_a[sl] = zeros16
    acc_b[sl] = zeros16

  lomask = jnp.full((16,), 0xFFFF, jnp.int32)
  himask = jnp.full((16,), -65536, jnp.int32)

  def process(eb, wbuf):
    @plsc.parallel_loop(0, VECS // 2, unroll=8)
    def _(h):
      for k, acc in ((0, acc_a), (1, acc_b)):
        sl = pl.ds(pl.multiple_of((h * 2 + k) * 16, 16), 16)
        a = eb[1, sl]
        w = wbuf[sl]
        packed = plsc.load_gather(packed_v, [a])
        d = plsc.bitcast(packed & himask, jnp.float32)
        s = packed & lomask
        plsc.addupdate_scatter(acc, [s], w * d)

  for g in range(MAX_G):
    eb, wbuf, si, sw = bufs[g % 2]
    pltpu.make_async_copy(ei_hbm.at[:, pl.ds(chunk_off(g), CHUNK)], eb, si).wait()
    pltpu.make_async_copy(w_hbm.at[pl.ds(chunk_off(g), CHUNK)], wbuf, sw).wait()
    if g + 1 < MAX_G:
      neb, nwb, nsi, nsw = bufs[(g + 1) % 2]
      start_dma(g + 1, neb, nwb, nsi, nsw)
    if g + 1 < MAX_G:
      process(eb, wbuf)
    else:
      @pl.when(wid + NUM_WORKERS * g < NCHUNKS)
      def _():
        process(eb, wbuf)

  @plsc.parallel_loop(0, NSUB_PAD // 16)
  def _(c):
    sl = pl.ds(pl.multiple_of(c * 16, 16), 16)
    s = acc_a[sl] + acc_b[sl]
    red_v[c // 8, pl.ds(pl.multiple_of((c % 8) * 16, 16), 16)] = s

  pltpu.sync_copy(red_v, out_hbm.at[wid])


def _combine_body(p_ref, y_ref, o_ref, *, nsub):
  s = jnp.sum(p_ref[...], axis=0)
  r = jnp.abs(s - y_ref[...])
  idx = (lax.broadcasted_iota(jnp.int32, (8, 128), 0) * 128
         + lax.broadcasted_iota(jnp.int32, (8, 128), 1))
  r = jnp.where(idx < nsub, r, 0.0)
  loss = jnp.sum(r) * (1.0 / nsub)
  o_ref[...] = jnp.zeros((8, 128), jnp.float32) + loss


def kernel(edge_weights, edge_index, agent_demand, agent_substation_map,
           substation_y_true, num_substations):
  submap = agent_substation_map.astype(jnp.uint32)
  nsub = substation_y_true.shape[0]

  d16 = lax.bitcast_convert_type(
      agent_demand.astype(jnp.bfloat16), jnp.uint16).astype(jnp.uint32)
  packed = lax.bitcast_convert_type((d16 << 16) | submap, jnp.int32)

  partials = _sc_scatter(edge_weights, edge_index.astype(jnp.int32), packed)

  y_pad = jnp.pad(substation_y_true, (0, NSUB_PAD - nsub)).reshape(8, 128)
  loss = pl.pallas_call(
      functools.partial(_combine_body, nsub=nsub),
      out_shape=jax.ShapeDtypeStruct((8, 128), jnp.float32),
  )(partials, y_pad)
  return loss[0, 0]


# speedup vs baseline: 1.1829x; 1.0066x over previous
import functools

import jax
import jax.numpy as jnp
from jax import lax
from jax.experimental import pallas as pl
from jax.experimental.pallas import tpu as pltpu
from jax.experimental.pallas import tpu_sc as plsc

N_NODES = 50000
N_EDGES = 1600000
NUM_WORKERS = 32
CHUNK = 12800
NCHUNKS = N_EDGES // CHUNK
MAX_G = -(-NCHUNKS // NUM_WORKERS)
VECS = CHUNK // 16
ACC_STRIDE = 1025
NSUB_PAD = 1024
LANES = 16

_mesh = plsc.VectorSubcoreMesh(core_axis_name="c", subcore_axis_name="s")


@functools.partial(
    pl.kernel,
    mesh=_mesh,
    compiler_params=pltpu.CompilerParams(needs_layout_passes=False),
    out_type=jax.ShapeDtypeStruct((NUM_WORKERS, 8, 128), jnp.float32),
    scratch_types=[
        pltpu.VMEM((N_NODES,), jnp.int32),
        pltpu.VMEM((2, CHUNK), jnp.int32),
        pltpu.VMEM((2, CHUNK), jnp.int32),
        pltpu.VMEM((CHUNK,), jnp.float32),
        pltpu.VMEM((CHUNK,), jnp.float32),
        pltpu.VMEM((NSUB_PAD,), jnp.float32),
        pltpu.VMEM((NSUB_PAD,), jnp.float32),
        pltpu.VMEM((8, 128), jnp.float32),
        pltpu.SemaphoreType.DMA,
        pltpu.SemaphoreType.DMA,
        pltpu.SemaphoreType.DMA,
        pltpu.SemaphoreType.DMA,
    ],
)
def _sc_scatter(w_hbm, ei_hbm, packed_hbm, out_hbm,
                packed_v, eb0, eb1, wb0, wb1, acc_a, acc_b, red_v,
                sem_i0, sem_i1, sem_w0, sem_w1):
  wid = lax.axis_index("c") * 16 + lax.axis_index("s")

  bufs = [(eb0, wb0, sem_i0, sem_w0), (eb1, wb1, sem_i1, sem_w1)]

  def chunk_off(g):
    cid = wid + NUM_WORKERS * g
    cid = jnp.minimum(cid, NCHUNKS - 1)
    return pl.multiple_of(cid * CHUNK, 128)

  def start_dma(g, eb, wbuf, si, sw):
    off = chunk_off(g)
    pltpu.make_async_copy(ei_hbm.at[:, pl.ds(off, CHUNK)], eb, si).start()
    pltpu.make_async_copy(w_hbm.at[pl.ds(off, CHUNK)], wbuf, sw).start()

  start_dma(0, eb0, wb0, sem_i0, sem_w0)
  pltpu.sync_copy(packed_hbm, packed_v)

  zeros16 = jnp.zeros((16,), jnp.float32)

  @plsc.parallel_loop(0, NSUB_PAD // 16)
  def _(i):
    sl = pl.ds(pl.multiple_of(i * 16, 16), 16)
    acc_a[sl] = zeros16
    acc_b[sl] = zeros16

  lomask = jnp.full((16,), 0xFFFF, jnp.int32)
  himask = jnp.full((16,), -65536, jnp.int32)

  def process(eb, wbuf):
    @plsc.parallel_loop(0, VECS // 2, unroll=4)
    def _(h):
      for k, acc in ((0, acc_a), (1, acc_b)):
        sl = pl.ds(pl.multiple_of((h * 2 + k) * 16, 16), 16)
        a = eb[1, sl]
        w = wbuf[sl]
        packed = plsc.load_gather(packed_v, [a])
        d = plsc.bitcast(packed & himask, jnp.float32)
        s = packed & lomask
        plsc.addupdate_scatter(acc, [s], w * d)

  for g in range(MAX_G):
    eb, wbuf, si, sw = bufs[g % 2]
    pltpu.make_async_copy(ei_hbm.at[:, pl.ds(chunk_off(g), CHUNK)], eb, si).wait()
    pltpu.make_async_copy(w_hbm.at[pl.ds(chunk_off(g), CHUNK)], wbuf, sw).wait()
    if g + 1 < MAX_G:
      neb, nwb, nsi, nsw = bufs[(g + 1) % 2]
      start_dma(g + 1, neb, nwb, nsi, nsw)
    if g + 1 < MAX_G:
      process(eb, wbuf)
    else:
      @pl.when(wid + NUM_WORKERS * g < NCHUNKS)
      def _():
        process(eb, wbuf)

  @plsc.parallel_loop(0, NSUB_PAD // 16)
  def _(c):
    sl = pl.ds(pl.multiple_of(c * 16, 16), 16)
    s = acc_a[sl] + acc_b[sl]
    red_v[c // 8, pl.ds(pl.multiple_of((c % 8) * 16, 16), 16)] = s

  pltpu.sync_copy(red_v, out_hbm.at[wid])


def _combine_body(p_ref, y_ref, o_ref, *, nsub):
  s = jnp.sum(p_ref[...], axis=0)
  r = jnp.abs(s - y_ref[...])
  idx = (lax.broadcasted_iota(jnp.int32, (8, 128), 0) * 128
         + lax.broadcasted_iota(jnp.int32, (8, 128), 1))
  r = jnp.where(idx < nsub, r, 0.0)
  loss = jnp.sum(r) * (1.0 / nsub)
  o_ref[...] = jnp.zeros((8, 128), jnp.float32) + loss


def kernel(edge_weights, edge_index, agent_demand, agent_substation_map,
           substation_y_true, num_substations):
  submap = agent_substation_map.astype(jnp.uint32)
  nsub = substation_y_true.shape[0]

  d16 = lax.bitcast_convert_type(
      agent_demand.astype(jnp.bfloat16), jnp.uint16).astype(jnp.uint32)
  packed = lax.bitcast_convert_type((d16 << 16) | submap, jnp.int32)

  partials = _sc_scatter(edge_weights, edge_index.astype(jnp.int32), packed)

  y_pad = jnp.pad(substation_y_true, (0, NSUB_PAD - nsub)).reshape(8, 128)
  loss = pl.pallas_call(
      functools.partial(_combine_body, nsub=nsub),
      out_shape=jax.ShapeDtypeStruct((8, 128), jnp.float32),
  )(partials, y_pad)
  return loss[0, 0]
